# baseline (speedup 1.0000x reference)
import jax
import jax.numpy as jnp
from jax.experimental import pallas as pl
from jax.experimental.pallas import tpu as pltpu

N = 16384
D = 512
C = 28
CP = 32
JB = 1024
IB = 256
NJ = N // JB
NIT = N // IB
TPB = JB // IB
RB = 2048
NB = N // RB
P = 1024

_HI = jax.lax.Precision.HIGHEST
_BIG = 2 ** 30


def _medoid_kernel(s1_ref, mci_ref, mcj_ref, cent_ref,
                   accj_ref, csa_ref, csb_ref):
    j = pl.program_id(0)

    @pl.when(j == 0)
    def _init():
        csb_ref[...] = jnp.zeros((NIT, CP, IB), jnp.float32)

    xj = s1_ref[pl.ds(j * JB, JB), :]
    sqj = jax.lax.dot_general(jnp.ones((1, D), jnp.float32), xj * xj,
                              (((1,), (1,)), ((), ())),
                              precision=_HI)
    masktj = mcj_ref[j]
    accj_ref[...] = jnp.zeros((CP, JB), jnp.float32)

    def tile_chain(t):
        i0 = t * IB
        xi = s1_ref[pl.ds(i0, IB), :]
        sqi = jnp.sum(xi * xi, axis=1, keepdims=True)
        g = jax.lax.dot_general(xi, xj, (((1,), (1,)), ((), ())))
        d2 = sqi + sqj - 2.0 * g
        dist = jnp.sqrt(jnp.maximum(d2, 0.0)
                        ).astype(jnp.bfloat16)
        maskti = mci_ref[t]
        u1 = jax.lax.dot_general(maskti, dist, (((1,), (0,)), ((), ())),
                                 preferred_element_type=jnp.float32)
        u2 = jax.lax.dot_general(masktj, dist, (((1,), (1,)), ((), ())),
                                 preferred_element_type=jnp.float32)
        return u1, u2

    def body(p, carry):
        res = [tile_chain(TPB * p + k) for k in range(TPB)]
        accj_ref[...] += sum(r[0] for r in res)

        @pl.when(p < j)
        def _lower():
            for k in range(TPB):
                csb_ref[TPB * p + k] += res[k][1]

        return carry

    jax.lax.fori_loop(0, j + 1, body, 0)

    for k in range(TPB):
        csa_ref[j * TPB + k] = accj_ref[:, k * IB:(k + 1) * IB]

    @pl.when(j == NJ - 1)
    def _finish():
        lane = jax.lax.broadcasted_iota(jnp.int32, (CP, IB), 1)

        def abody(t, carry):
            mv, mi = carry
            cs = csa_ref[t] + csb_ref[t]
            m = jnp.where(mci_ref[t] > 0, cs, jnp.inf)
            bm = jnp.min(m, axis=1, keepdims=True)
            bi = jnp.min(jnp.where(m == bm, lane + t * IB, _BIG),
                         axis=1, keepdims=True)
            upd = bm < mv
            return (jnp.where(upd, bm, mv), jnp.where(upd, bi, mi))

        _, amin = jax.lax.fori_loop(
            0, NIT, abody,
            (jnp.full((CP, 1), jnp.inf, jnp.float32),
             jnp.zeros((CP, 1), jnp.int32)))

        def gbody(t, cacc):
            i0 = t * IB
            xi = s1_ref[pl.ds(i0, IB), :]
            sel = (lane + i0 == amin).astype(jnp.float32)
            return cacc + jax.lax.dot_general(
                sel, xi, (((1,), (0,)), ((), ())), precision=_HI)

        cent = jax.lax.fori_loop(0, NIT, gbody,
                                 jnp.zeros((CP, D), jnp.float32))
        cent_ref[...] = cent


def _loss_kernel(s1_ref, s2_ref, segc_ref, sim_ref, cent_ref, out_ref,
                 accl_ref, accg_ref):
    b = pl.program_id(0)

    @pl.when(b == 0)
    def _init():
        accl_ref[...] = jnp.zeros((1, 1), jnp.float32)
        accg_ref[...] = jnp.zeros((1, 1), jnp.float32)

    w = jnp.sum(sim_ref[...], axis=1, keepdims=True) * (1.0 / P)
    centb = cent_ref[...].astype(jnp.bfloat16)
    col = jax.lax.broadcasted_iota(jnp.int32, (RB, CP), 1)
    valid = col < C
    seg = segc_ref[...]
    inv_t = jnp.float32(1.0 / 0.07)

    def ce_weighted(x_ref):
        logits = jax.lax.dot_general(
            x_ref[...], centb, (((1,), (1,)), ((), ())),
            preferred_element_type=jnp.float32) * inv_t
        lm = jnp.where(valid, logits, -jnp.inf)
        m = jnp.max(lm, axis=1, keepdims=True)
        lse = m + jnp.log(jnp.sum(jnp.exp(lm - m), axis=1, keepdims=True))
        picked = jnp.sum(jnp.where(col == seg, logits, 0.0),
                         axis=1, keepdims=True)
        return jnp.sum((lse - picked) * w)

    accl_ref[...] += ce_weighted(s1_ref).reshape(1, 1)
    accg_ref[...] += ce_weighted(s2_ref).reshape(1, 1)

    @pl.when(b == NB - 1)
    def _fin():
        local = accl_ref[0, 0] * (1.0 / N)
        glob = accg_ref[0, 0] * (1.0 / N)
        out_ref[...] = (((1.0 - 0.7) * local + 0.7 * glob) * 0.5
                        ).reshape(1, 1)


@jax.jit
def kernel(S1, S2, segmentation_map, similarity_matrix,
           learned_centroids, prototypes):
    del learned_centroids, prototypes
    seg = segmentation_map.reshape(-1)
    segcol = seg.reshape(N, 1)
    sim2d = similarity_matrix.reshape(N, P)
    S1b = S1.astype(jnp.bfloat16)
    S2b = S2.astype(jnp.bfloat16)
    maskc = (jnp.arange(CP, dtype=seg.dtype)[:, None] == seg[None, :]
             ).astype(jnp.bfloat16)
    mci = maskc.reshape(CP, NIT, IB).transpose(1, 0, 2)
    mcj = maskc.reshape(CP, NJ, JB).transpose(1, 0, 2)

    cent = pl.pallas_call(
        _medoid_kernel,
        grid=(NJ,),
        in_specs=[
            pl.BlockSpec((N, D), lambda j: (0, 0)),
            pl.BlockSpec((NIT, CP, IB), lambda j: (0, 0, 0)),
            pl.BlockSpec((NJ, CP, JB), lambda j: (0, 0, 0)),
        ],
        out_specs=pl.BlockSpec((CP, D), lambda j: (0, 0)),
        out_shape=jax.ShapeDtypeStruct((CP, D), jnp.float32),
        scratch_shapes=[
            pltpu.VMEM((CP, JB), jnp.float32),
            pltpu.VMEM((NIT, CP, IB), jnp.float32),
            pltpu.VMEM((NIT, CP, IB), jnp.float32),
        ],
        compiler_params=pltpu.CompilerParams(
            vmem_limit_bytes=63 * 1024 * 1024),
    )(S1, mci, mcj)

    out = pl.pallas_call(
        _loss_kernel,
        grid=(NB,),
        in_specs=[
            pl.BlockSpec((RB, D), lambda b: (b, 0)),
            pl.BlockSpec((RB, D), lambda b: (b, 0)),
            pl.BlockSpec((RB, 1), lambda b: (b, 0)),
            pl.BlockSpec((RB, P), lambda b: (b, 0)),
            pl.BlockSpec((CP, D), lambda b: (0, 0)),
        ],
        out_specs=pl.BlockSpec((1, 1), lambda b: (0, 0)),
        out_shape=jax.ShapeDtypeStruct((1, 1), jnp.float32),
        scratch_shapes=[
            pltpu.VMEM((1, 1), jnp.float32),
            pltpu.VMEM((1, 1), jnp.float32),
        ],
    )(S1b, S2b, segcol, sim2d, cent)

    return out.reshape(())


# speedup vs baseline: 1.0814x; 1.0814x over previous
import jax
import jax.numpy as jnp
from jax.experimental import pallas as pl
from jax.experimental.pallas import tpu as pltpu

N = 16384
D = 512
C = 28
CP = 32
JB = 2048
IB = 512
NJ = N // JB
NIT = N // IB
TPB = JB // IB
RB = 2048
NB = N // RB
P = 1024

_HI = jax.lax.Precision.HIGHEST
_BIG = 2 ** 30


def _medoid_kernel(s1_ref, mci_ref, mcj_ref, cent_ref,
                   accj_ref, csa_ref, csb_ref):
    j = pl.program_id(0)

    @pl.when(j == 0)
    def _init():
        csb_ref[...] = jnp.zeros((NIT, CP, IB), jnp.float32)

    xj = s1_ref[pl.ds(j * JB, JB), :]
    sqj = jax.lax.dot_general(jnp.ones((1, D), jnp.float32), xj * xj,
                              (((1,), (1,)), ((), ())),
                              precision=_HI)
    masktj = mcj_ref[j]
    accj_ref[...] = jnp.zeros((CP, JB), jnp.float32)

    def tile_chain(t):
        i0 = t * IB
        xi = s1_ref[pl.ds(i0, IB), :]
        sqi = jnp.sum(xi * xi, axis=1, keepdims=True)
        g = jax.lax.dot_general(xi, xj, (((1,), (1,)), ((), ())))
        d2 = sqi + sqj - 2.0 * g
        dist = jnp.sqrt(jnp.maximum(d2, 0.0)
                        ).astype(jnp.bfloat16)
        maskti = mci_ref[t]
        u1 = jax.lax.dot_general(maskti, dist, (((1,), (0,)), ((), ())),
                                 preferred_element_type=jnp.float32)
        u2 = jax.lax.dot_general(masktj, dist, (((1,), (1,)), ((), ())),
                                 preferred_element_type=jnp.float32)
        return u1, u2

    def body(p, carry):
        res = [tile_chain(TPB * p + k) for k in range(TPB)]
        accj_ref[...] += sum(r[0] for r in res)

        @pl.when(p < j)
        def _lower():
            for k in range(TPB):
                csb_ref[TPB * p + k] += res[k][1]

        return carry

    jax.lax.fori_loop(0, j + 1, body, 0)

    for k in range(TPB):
        csa_ref[j * TPB + k] = accj_ref[:, k * IB:(k + 1) * IB]

    @pl.when(j == NJ - 1)
    def _finish():
        lane = jax.lax.broadcasted_iota(jnp.int32, (CP, IB), 1)

        def abody(t, carry):
            mv, mi = carry
            cs = csa_ref[t] + csb_ref[t]
            m = jnp.where(mci_ref[t] > 0, cs, jnp.inf)
            bm = jnp.min(m, axis=1, keepdims=True)
            bi = jnp.min(jnp.where(m == bm, lane + t * IB, _BIG),
                         axis=1, keepdims=True)
            upd = bm < mv
            return (jnp.where(upd, bm, mv), jnp.where(upd, bi, mi))

        _, amin = jax.lax.fori_loop(
            0, NIT, abody,
            (jnp.full((CP, 1), jnp.inf, jnp.float32),
             jnp.zeros((CP, 1), jnp.int32)))

        def gbody(t, cacc):
            i0 = t * IB
            xi = s1_ref[pl.ds(i0, IB), :]
            sel = (lane + i0 == amin).astype(jnp.float32)
            return cacc + jax.lax.dot_general(
                sel, xi, (((1,), (0,)), ((), ())), precision=_HI)

        cent = jax.lax.fori_loop(0, NIT, gbody,
                                 jnp.zeros((CP, D), jnp.float32))
        cent_ref[...] = cent


def _loss_kernel(s1_ref, s2_ref, segc_ref, sim_ref, cent_ref, out_ref,
                 accl_ref, accg_ref):
    b = pl.program_id(0)

    @pl.when(b == 0)
    def _init():
        accl_ref[...] = jnp.zeros((1, 1), jnp.float32)
        accg_ref[...] = jnp.zeros((1, 1), jnp.float32)

    w = jnp.sum(sim_ref[...], axis=1, keepdims=True) * (1.0 / P)
    centb = cent_ref[...].astype(jnp.bfloat16)
    col = jax.lax.broadcasted_iota(jnp.int32, (RB, CP), 1)
    valid = col < C
    seg = segc_ref[...]
    inv_t = jnp.float32(1.0 / 0.07)

    def ce_weighted(x_ref):
        logits = jax.lax.dot_general(
            x_ref[...], centb, (((1,), (1,)), ((), ())),
            preferred_element_type=jnp.float32) * inv_t
        lm = jnp.where(valid, logits, -jnp.inf)
        m = jnp.max(lm, axis=1, keepdims=True)
        lse = m + jnp.log(jnp.sum(jnp.exp(lm - m), axis=1, keepdims=True))
        picked = jnp.sum(jnp.where(col == seg, logits, 0.0),
                         axis=1, keepdims=True)
        return jnp.sum((lse - picked) * w)

    accl_ref[...] += ce_weighted(s1_ref).reshape(1, 1)
    accg_ref[...] += ce_weighted(s2_ref).reshape(1, 1)

    @pl.when(b == NB - 1)
    def _fin():
        local = accl_ref[0, 0] * (1.0 / N)
        glob = accg_ref[0, 0] * (1.0 / N)
        out_ref[...] = (((1.0 - 0.7) * local + 0.7 * glob) * 0.5
                        ).reshape(1, 1)


@jax.jit
def kernel(S1, S2, segmentation_map, similarity_matrix,
           learned_centroids, prototypes):
    del learned_centroids, prototypes
    seg = segmentation_map.reshape(-1)
    segcol = seg.reshape(N, 1)
    sim2d = similarity_matrix.reshape(N, P)
    S1b = S1.astype(jnp.bfloat16)
    S2b = S2.astype(jnp.bfloat16)
    maskc = (jnp.arange(CP, dtype=seg.dtype)[:, None] == seg[None, :]
             ).astype(jnp.bfloat16)
    mci = maskc.reshape(CP, NIT, IB).transpose(1, 0, 2)
    mcj = maskc.reshape(CP, NJ, JB).transpose(1, 0, 2)

    cent = pl.pallas_call(
        _medoid_kernel,
        grid=(NJ,),
        in_specs=[
            pl.BlockSpec((N, D), lambda j: (0, 0)),
            pl.BlockSpec((NIT, CP, IB), lambda j: (0, 0, 0)),
            pl.BlockSpec((NJ, CP, JB), lambda j: (0, 0, 0)),
        ],
        out_specs=pl.BlockSpec((CP, D), lambda j: (0, 0)),
        out_shape=jax.ShapeDtypeStruct((CP, D), jnp.float32),
        scratch_shapes=[
            pltpu.VMEM((CP, JB), jnp.float32),
            pltpu.VMEM((NIT, CP, IB), jnp.float32),
            pltpu.VMEM((NIT, CP, IB), jnp.float32),
        ],
        compiler_params=pltpu.CompilerParams(
            vmem_limit_bytes=63 * 1024 * 1024),
    )(S1, mci, mcj)

    out = pl.pallas_call(
        _loss_kernel,
        grid=(NB,),
        in_specs=[
            pl.BlockSpec((RB, D), lambda b: (b, 0)),
            pl.BlockSpec((RB, D), lambda b: (b, 0)),
            pl.BlockSpec((RB, 1), lambda b: (b, 0)),
            pl.BlockSpec((RB, P), lambda b: (b, 0)),
            pl.BlockSpec((CP, D), lambda b: (0, 0)),
        ],
        out_specs=pl.BlockSpec((1, 1), lambda b: (0, 0)),
        out_shape=jax.ShapeDtypeStruct((1, 1), jnp.float32),
        scratch_shapes=[
            pltpu.VMEM((1, 1), jnp.float32),
            pltpu.VMEM((1, 1), jnp.float32),
        ],
    )(S1b, S2b, segcol, sim2d, cent)

    return out.reshape(())


# speedup vs baseline: 1.1445x; 1.0584x over previous
import jax
import jax.numpy as jnp
from jax.experimental import pallas as pl
from jax.experimental.pallas import tpu as pltpu

N = 16384
D = 512
C = 28
CP = 32
JB = 2048
IB = 512
NJ = N // JB
NIT = N // IB
TPB = JB // IB
RB = 2048
NB = N // RB
P = 1024

_HI = jax.lax.Precision.HIGHEST
_BIG = 2 ** 30


def _medoid_kernel(s1_ref, mci_ref, mcj_ref, cent_ref,
                   accj_ref, csa_ref, csb_ref):
    j = pl.program_id(0)

    @pl.when(j == 0)
    def _init():
        csb_ref[...] = jnp.zeros((NIT, CP, IB), jnp.float32)

    xj = s1_ref[pl.ds(j * JB, JB), :]
    sqj = jax.lax.dot_general(jnp.ones((1, D), jnp.float32), xj * xj,
                              (((1,), (1,)), ((), ())),
                              precision=_HI)
    masktj = mcj_ref[j]
    accj_ref[...] = jnp.zeros((CP, JB), jnp.float32)

    def tile_chain(t):
        i0 = t * IB
        xi = s1_ref[pl.ds(i0, IB), :]
        sqi = jnp.sum(xi * xi, axis=1, keepdims=True)
        g = jax.lax.dot_general(xi, xj, (((1,), (1,)), ((), ())))
        d2 = sqi + sqj - 2.0 * g
        dist = jnp.sqrt(jnp.maximum(d2, 0.0)
                        ).astype(jnp.bfloat16)
        maskti = mci_ref[t]
        u1 = jax.lax.dot_general(maskti, dist, (((1,), (0,)), ((), ())),
                                 preferred_element_type=jnp.float32)
        return u1, dist

    def body(p, carry):
        res = [tile_chain(TPB * p + k) for k in range(TPB)]
        accj_ref[...] += sum(r[0] for r in res)

        @pl.when(p < j)
        def _lower():
            for k in range(TPB):
                u2 = jax.lax.dot_general(masktj, res[k][1],
                                         (((1,), (1,)), ((), ())),
                                         preferred_element_type=jnp.float32)
                csb_ref[TPB * p + k] += u2

        return carry

    jax.lax.fori_loop(0, j + 1, body, 0)

    for k in range(TPB):
        csa_ref[j * TPB + k] = accj_ref[:, k * IB:(k + 1) * IB]

    @pl.when(j == NJ - 1)
    def _finish():
        lane = jax.lax.broadcasted_iota(jnp.int32, (CP, IB), 1)

        def abody(t, carry):
            mv, mi = carry
            cs = csa_ref[t] + csb_ref[t]
            m = jnp.where(mci_ref[t] > 0, cs, jnp.inf)
            bm = jnp.min(m, axis=1, keepdims=True)
            bi = jnp.min(jnp.where(m == bm, lane + t * IB, _BIG),
                         axis=1, keepdims=True)
            upd = bm < mv
            return (jnp.where(upd, bm, mv), jnp.where(upd, bi, mi))

        _, amin = jax.lax.fori_loop(
            0, NIT, abody,
            (jnp.full((CP, 1), jnp.inf, jnp.float32),
             jnp.zeros((CP, 1), jnp.int32)))

        def gbody(t, cacc):
            i0 = t * IB
            xi = s1_ref[pl.ds(i0, IB), :]
            sel = (lane + i0 == amin).astype(jnp.float32)
            return cacc + jax.lax.dot_general(
                sel, xi, (((1,), (0,)), ((), ())), precision=_HI)

        cent = jax.lax.fori_loop(0, NIT, gbody,
                                 jnp.zeros((CP, D), jnp.float32))
        cent_ref[...] = cent


def _loss_kernel(s1_ref, s2_ref, segc_ref, sim_ref, cent_ref, out_ref,
                 accl_ref, accg_ref):
    b = pl.program_id(0)

    @pl.when(b == 0)
    def _init():
        accl_ref[...] = jnp.zeros((1, 1), jnp.float32)
        accg_ref[...] = jnp.zeros((1, 1), jnp.float32)

    w = jnp.sum(sim_ref[...], axis=1, keepdims=True) * (1.0 / P)
    centb = cent_ref[...].astype(jnp.bfloat16)
    col = jax.lax.broadcasted_iota(jnp.int32, (RB, CP), 1)
    valid = col < C
    seg = segc_ref[...]
    inv_t = jnp.float32(1.0 / 0.07)

    def ce_weighted(x_ref):
        logits = jax.lax.dot_general(
            x_ref[...], centb, (((1,), (1,)), ((), ())),
            preferred_element_type=jnp.float32) * inv_t
        lm = jnp.where(valid, logits, -jnp.inf)
        m = jnp.max(lm, axis=1, keepdims=True)
        lse = m + jnp.log(jnp.sum(jnp.exp(lm - m), axis=1, keepdims=True))
        picked = jnp.sum(jnp.where(col == seg, logits, 0.0),
                         axis=1, keepdims=True)
        return jnp.sum((lse - picked) * w)

    accl_ref[...] += ce_weighted(s1_ref).reshape(1, 1)
    accg_ref[...] += ce_weighted(s2_ref).reshape(1, 1)

    @pl.when(b == NB - 1)
    def _fin():
        local = accl_ref[0, 0] * (1.0 / N)
        glob = accg_ref[0, 0] * (1.0 / N)
        out_ref[...] = (((1.0 - 0.7) * local + 0.7 * glob) * 0.5
                        ).reshape(1, 1)


@jax.jit
def kernel(S1, S2, segmentation_map, similarity_matrix,
           learned_centroids, prototypes):
    del learned_centroids, prototypes
    seg = segmentation_map.reshape(-1)
    segcol = seg.reshape(N, 1)
    sim2d = similarity_matrix.reshape(N, P)
    S1b = S1.astype(jnp.bfloat16)
    S2b = S2.astype(jnp.bfloat16)
    maskc = (jnp.arange(CP, dtype=seg.dtype)[:, None] == seg[None, :]
             ).astype(jnp.bfloat16)
    mci = maskc.reshape(CP, NIT, IB).transpose(1, 0, 2)
    mcj = maskc.reshape(CP, NJ, JB).transpose(1, 0, 2)

    cent = pl.pallas_call(
        _medoid_kernel,
        grid=(NJ,),
        in_specs=[
            pl.BlockSpec((N, D), lambda j: (0, 0)),
            pl.BlockSpec((NIT, CP, IB), lambda j: (0, 0, 0)),
            pl.BlockSpec((NJ, CP, JB), lambda j: (0, 0, 0)),
        ],
        out_specs=pl.BlockSpec((CP, D), lambda j: (0, 0)),
        out_shape=jax.ShapeDtypeStruct((CP, D), jnp.float32),
        scratch_shapes=[
            pltpu.VMEM((CP, JB), jnp.float32),
            pltpu.VMEM((NIT, CP, IB), jnp.float32),
            pltpu.VMEM((NIT, CP, IB), jnp.float32),
        ],
        compiler_params=pltpu.CompilerParams(
            vmem_limit_bytes=63 * 1024 * 1024),
    )(S1, mci, mcj)

    out = pl.pallas_call(
        _loss_kernel,
        grid=(NB,),
        in_specs=[
            pl.BlockSpec((RB, D), lambda b: (b, 0)),
            pl.BlockSpec((RB, D), lambda b: (b, 0)),
            pl.BlockSpec((RB, 1), lambda b: (b, 0)),
            pl.BlockSpec((RB, P), lambda b: (b, 0)),
            pl.BlockSpec((CP, D), lambda b: (0, 0)),
        ],
        out_specs=pl.BlockSpec((1, 1), lambda b: (0, 0)),
        out_shape=jax.ShapeDtypeStruct((1, 1), jnp.float32),
        scratch_shapes=[
            pltpu.VMEM((1, 1), jnp.float32),
            pltpu.VMEM((1, 1), jnp.float32),
        ],
    )(S1b, S2b, segcol, sim2d, cent)

    return out.reshape(())
